# Initial kernel scaffold; baseline (speedup 1.0000x reference)
import jax
import jax.numpy as jnp
from jax.experimental import pallas as pl


def kernel(x, edge_index, W1, b1, W2, b2):
    raise NotImplementedError("write your pallas kernel here")


# speedup vs baseline: 18.7267x; 18.7267x over previous
import functools

import jax
import jax.numpy as jnp
from jax import lax
from jax.experimental import pallas as pl
from jax.experimental.pallas import tpu as pltpu
from jax.experimental.pallas import tpu_sc as plsc

_NC = 2
_NS = 16
_NW = _NC * _NS


def _rows_per_tile(n):
    return 128 * (-(-n // (_NS * 128)))


def _make_deg_kernel(n, ch, k):
    rpt = _rows_per_tile(n)
    npad = rpt * _NS
    nz = rpt // k
    mesh = plsc.VectorSubcoreMesh(core_axis_name="c", subcore_axis_name="s")

    @functools.partial(
        pl.kernel, mesh=mesh,
        out_type=jax.ShapeDtypeStruct((_NC, npad, 16), jnp.float32),
        scratch_types=[
            pltpu.VMEM((ch, k), jnp.int32),
            pltpu.VMEM((k, 16), jnp.float32),
            pltpu.VMEM_SHARED((npad, 16), jnp.float32),
        ],
    )
    def deg_k(dst_hbm, out_hbm, dstv, onev, acc):
        c = lax.axis_index("c")
        s = lax.axis_index("s")
        w = c * _NS + s

        def zrow(i, carry):
            onev[i, :] = jnp.zeros((16,), jnp.float32)
            return carry

        lax.fori_loop(0, k, zrow, 0)
        for m in range(nz):
            pltpu.sync_copy(onev, acc.at[pl.ds(s * rpt + m * k, k), :])

        def orow(i, carry):
            onev[i, :] = jnp.ones((16,), jnp.float32)
            return carry

        lax.fori_loop(0, k, orow, 0)
        pltpu.sync_copy(dst_hbm.at[w], dstv)
        plsc.subcore_barrier()

        def chunk(j, carry):
            pltpu.sync_copy(onev, acc.at[dstv.at[j]], add=True)
            return carry

        lax.fori_loop(0, ch, chunk, 0)
        plsc.subcore_barrier()
        for m in range(nz):
            pltpu.sync_copy(acc.at[pl.ds(s * rpt + m * k, k), :], onev)
            pltpu.sync_copy(onev, out_hbm.at[c, pl.ds(s * rpt + m * k, k), :])

    return deg_k


def _make_agg_kernel(n, d, ch, k):
    rpt = _rows_per_tile(n)
    npad = rpt * _NS
    nz = rpt // k
    mesh = plsc.VectorSubcoreMesh(core_axis_name="c", subcore_axis_name="s")

    @functools.partial(
        pl.kernel, mesh=mesh,
        out_type=jax.ShapeDtypeStruct((_NC, npad, d), jnp.float32),
        scratch_types=[
            pltpu.VMEM((ch, k), jnp.int32),
            pltpu.VMEM((ch, k), jnp.int32),
            pltpu.VMEM((k, d), jnp.float32),
            pltpu.VMEM_SHARED((npad, d), jnp.float32),
            pltpu.SemaphoreType.DMA,
        ],
    )
    def agg_k(g_hbm, src_hbm, dst_hbm, out_hbm, srcv, dstv, rows, acc, sem):
        c = lax.axis_index("c")
        s = lax.axis_index("s")
        w = c * _NS + s

        def zrow(i, carry):
            for t in range(d // 16):
                rows[i, pl.ds(t * 16, 16)] = jnp.zeros((16,), jnp.float32)
            return carry

        lax.fori_loop(0, k, zrow, 0)
        for m in range(nz):
            pltpu.sync_copy(rows, acc.at[pl.ds(s * rpt + m * k, k), :])

        pltpu.sync_copy(src_hbm.at[w], srcv)
        pltpu.sync_copy(dst_hbm.at[w], dstv)
        plsc.subcore_barrier()

        def chunk(j, carry):
            pltpu.async_copy(g_hbm.at[srcv.at[j]], rows, sem).wait()
            pltpu.sync_copy(rows, acc.at[dstv.at[j]], add=True)
            return carry

        lax.fori_loop(0, ch, chunk, 0)
        plsc.subcore_barrier()
        for m in range(nz):
            pltpu.sync_copy(acc.at[pl.ds(s * rpt + m * k, k), :], rows)
            pltpu.sync_copy(rows, out_hbm.at[c, pl.ds(s * rpt + m * k, k), :])

    return agg_k


def _dinv_from(degp):
    deg = 1.0 + degp[0][:, 0:1] + degp[1][:, 0:1]
    return lax.rsqrt(deg)


def _mm_scale_body(x_ref, w_ref, degp_ref, o_ref):
    dinv = _dinv_from(degp_ref[...])
    o_ref[...] = jnp.dot(
        x_ref[...], w_ref[...], preferred_element_type=jnp.float32) * dinv


def _mid_body(p_ref, g1_ref, degp_ref, b1_ref, w2_ref, o_ref):
    dinv = _dinv_from(degp_ref[...])
    p = p_ref[...]
    h = (p[0] + p[1] + g1_ref[...]) * dinv + b1_ref[...]
    h = jnp.maximum(h, 0.0)
    o_ref[...] = jnp.dot(
        h, w2_ref[...], preferred_element_type=jnp.float32) * dinv


def _fin_body(q_ref, g2_ref, degp_ref, b2_ref, o_ref):
    dinv = _dinv_from(degp_ref[...])
    q = q_ref[...]
    o_ref[...] = (q[0] + q[1] + g2_ref[...]) * dinv + b2_ref[...]


def kernel(x, edge_index, W1, b1, W2, b2):
    n, d_in = x.shape
    d_hid = W1.shape[1]
    d_out = W2.shape[1]
    e = edge_index.shape[1]
    k = 80
    ch = e // (k * _NW)
    r = 400

    src3d = edge_index[0].reshape(_NW, ch, k)
    dst3d = edge_index[1].reshape(_NW, ch, k)

    deg_k = _make_deg_kernel(n, ch, k)
    agg_hid = _make_agg_kernel(n, d_hid, ch, k)

    degp = deg_k(dst3d)

    full = lambda *shape: pl.BlockSpec(shape, lambda i: (0,) * len(shape))
    rows = lambda *shape: pl.BlockSpec((r,) + shape, lambda i: (i,) + (0,) * len(shape))
    degs = pl.BlockSpec((2, r, 16), lambda i: (0, i, 0))
    prt = lambda dd: pl.BlockSpec((2, r, dd), lambda i: (0, i, 0))

    g1 = pl.pallas_call(
        _mm_scale_body,
        grid=(n // r,),
        in_specs=[rows(d_in), full(d_in, d_hid), degs],
        out_specs=rows(d_hid),
        out_shape=jax.ShapeDtypeStruct((n, d_hid), jnp.float32),
    )(x, W1, degp)

    p = agg_hid(g1, src3d, dst3d)

    g2 = pl.pallas_call(
        _mid_body,
        grid=(n // r,),
        in_specs=[prt(d_hid), rows(d_hid), degs, full(1, d_hid), full(d_hid, d_out)],
        out_specs=rows(d_out),
        out_shape=jax.ShapeDtypeStruct((n, d_out), jnp.float32),
    )(p, g1, degp, b1.reshape(1, d_hid), W2)

    if d_out == d_hid:
        agg_out = agg_hid
    else:
        agg_out = _make_agg_kernel(n, d_out, ch, k)
    q = agg_out(g2, src3d, dst3d)

    out = pl.pallas_call(
        _fin_body,
        grid=(n // r,),
        in_specs=[prt(d_out), rows(d_out), degs, full(1, d_out)],
        out_specs=rows(d_out),
        out_shape=jax.ShapeDtypeStruct((n, d_out), jnp.float32),
    )(q, g2, degp, b2.reshape(1, d_out))

    return out
